# baseline (speedup 1.0000x reference)
import jax
import jax.numpy as jnp
from jax import lax
from jax.experimental import pallas as pl
from jax.experimental.pallas import tpu as pltpu
from jax.experimental.pallas import tpu_sc as plsc

_B = 16384
_D = 32
_L = 16
_RW = 128
_N = 1_000_000
_TCBLK = 1024
_TCGRID = -(-_N // _TCBLK)
_ROWS = _TCGRID * (_TCBLK // 4)

_INFO = plsc.get_sparse_core_info()
_NC = _INFO.num_cores
_NS = _INFO.num_subcores
_NW = _NC * _NS
_BPW = _B // _NW
_HALF = _BPW // 2
_HGROUPS = _HALF // _L
_CHUNK = 128
_NHCHUNK = _HALF // _CHUNK
_NCHUNK = _BPW // _CHUNK


def _tc_body(x_ref, o_ref):
    x = x_ref[...]
    for c in range(4):
        o_ref[:, c * _D:(c + 1) * _D] = x[:, c * 256:(c + 1) * 256].T


def _sc_body(idx1_hbm, idx2_hbm, emb4_hbm, reff_hbm, beta_hbm, out_hbm,
             idx1_v, idx2_v, k1_v, k2_v, o1_v, o2_v,
             zrow1_v, zrow2_v, r1_v, r2_v, beta_v, out_v, sem, rsem):
    wid = lax.axis_index("s") * _NC + lax.axis_index("c")
    base = wid * _BPW

    pltpu.sync_copy(idx1_hbm.at[pl.ds(base, _BPW)], idx1_v)
    pltpu.sync_copy(idx2_hbm.at[pl.ds(base, _BPW)], idx2_v)
    pltpu.sync_copy(beta_hbm, beta_v)

    rcopies = []
    for k in range(_NCHUNK):
        s = pl.ds(k * _CHUNK, _CHUNK)
        rcopies.append(pltpu.async_copy(reff_hbm.at[idx1_v.at[s]], r1_v.at[s], rsem))
        rcopies.append(pltpu.async_copy(reff_hbm.at[idx2_v.at[s]], r2_v.at[s], rsem))

    def prep(g, carry):
        s = pl.ds(g * _L, _L)
        i1 = idx1_v[s]
        i2 = idx2_v[s]
        k1_v[s] = ((i1 >> 10) << 8) + (i1 & 255)
        k2_v[s] = ((i2 >> 10) << 8) + (i2 & 255)
        o1_v[s] = ((i1 >> 8) & 3) << 5
        o2_v[s] = ((i2 >> 8) & 3) << 5
        return carry

    lax.fori_loop(0, _BPW // _L, prep, 0)

    beta = beta_v[...]
    lane = lax.iota(jnp.int32, _L)

    def issue(q):
        par = q & 1
        s = pl.ds(q * _CHUNK, _CHUNK)
        d = pl.ds(par * _CHUNK, _CHUNK)
        return [
            pltpu.async_copy(emb4_hbm.at[k1_v.at[s]], zrow1_v.at[d, :], sem),
            pltpu.async_copy(emb4_hbm.at[k2_v.at[s]], zrow2_v.at[d, :], sem),
        ]

    def compute_quarter(q):
        par = q & 1

        def group(g, carry2):
            sl = pl.ds(q * _CHUNK + g * _L, _L)
            rows = par * _CHUNK + g * _L + lane
            o1 = o1_v[sl]
            o2 = o2_v[sl]
            acc = jnp.zeros((_L,), jnp.float32)
            for d in range(_D):
                a = plsc.load_gather(zrow1_v, [rows, o1 + d])
                b = plsc.load_gather(zrow2_v, [rows, o2 + d])
                diff = a - b
                acc = acc + diff * diff
            i = plsc.bitcast(acc, jnp.int32)
            i = jnp.int32(0x5F3759DF) - (i >> 1)
            y = plsc.bitcast(i, jnp.float32)
            for _ in range(3):
                y = y * (1.5 - 0.5 * acc * y * y)
            dist = jnp.where(acc > 1e-35, acc * y, 0.0)
            out_v[sl] = r1_v[sl] + r2_v[sl] - beta * dist
            return carry2

        lax.fori_loop(0, _CHUNK // _L, group, 0)

    nq = _BPW // _CHUNK
    pending = issue(0)
    for c in rcopies:
        c.wait()
    for q in range(nq):
        for c in pending:
            c.wait()
        if q + 1 < nq:
            nxt = issue(q + 1)
        else:
            nxt = []
        compute_quarter(q)
        pending = nxt
    pltpu.sync_copy(out_v, out_hbm.at[pl.ds(base, _BPW)])


@jax.jit
def _run(p1, p2, embt, reff_flat, beta16):
    emb4 = pl.pallas_call(
        _tc_body,
        grid=(_TCGRID,),
        in_specs=[pl.BlockSpec((_D, _TCBLK), lambda j: (0, j))],
        out_specs=pl.BlockSpec((_TCBLK // 4, _RW), lambda j: (j, 0)),
        out_shape=jax.ShapeDtypeStruct((_ROWS, _RW), jnp.float32),
    )(embt)

    ker = pl.kernel(
        _sc_body,
        out_type=jax.ShapeDtypeStruct((_B,), jnp.float32),
        mesh=plsc.VectorSubcoreMesh(core_axis_name="c", subcore_axis_name="s"),
        compiler_params=pltpu.CompilerParams(
            needs_layout_passes=False, use_tc_tiling_on_sc=True),
        scratch_types=[
            pltpu.VMEM((_BPW,), jnp.int32),
            pltpu.VMEM((_BPW,), jnp.int32),
            pltpu.VMEM((_BPW,), jnp.int32),
            pltpu.VMEM((_BPW,), jnp.int32),
            pltpu.VMEM((_BPW,), jnp.int32),
            pltpu.VMEM((_BPW,), jnp.int32),
            pltpu.VMEM((2 * _CHUNK, _RW), jnp.float32),
            pltpu.VMEM((2 * _CHUNK, _RW), jnp.float32),
            pltpu.VMEM((_BPW,), jnp.float32),
            pltpu.VMEM((_BPW,), jnp.float32),
            pltpu.VMEM((_L,), jnp.float32),
            pltpu.VMEM((_BPW,), jnp.float32),
            pltpu.SemaphoreType.DMA,
            pltpu.SemaphoreType.DMA,
        ],
    )
    return ker(p1, p2, emb4, reff_flat, beta16)


def kernel(protein1_idx, protein2_idx, embeddings, random_effects, beta):
    p1 = protein1_idx.astype(jnp.int32)
    p2 = protein2_idx.astype(jnp.int32)
    embt = embeddings.T
    reff_flat = random_effects.reshape(-1)
    beta16 = jnp.full((_L,), beta, jnp.float32)
    return _run(p1, p2, embt, reff_flat, beta16)


# speedup vs baseline: 3.3064x; 3.3064x over previous
import jax
import jax.numpy as jnp
from jax import lax
from jax.experimental import pallas as pl
from jax.experimental.pallas import tpu as pltpu
from jax.experimental.pallas import tpu_sc as plsc

_B = 16384
_D = 32
_L = 16
_RW = 128
_N = 1_000_000
_TCBLK = 8192
_TCGRID = -(-_N // _TCBLK)
_ROWS = _TCGRID * (_TCBLK // 4)

_INFO = plsc.get_sparse_core_info()
_NC = _INFO.num_cores
_NS = _INFO.num_subcores
_NW = _NC * _NS
_BPW = _B // _NW
_HALF = _BPW // 2
_HGROUPS = _HALF // _L
_CHUNK = 128
_NHCHUNK = _HALF // _CHUNK
_NCHUNK = _BPW // _CHUNK


def _tc_body(x_ref, o_ref):
    x = x_ref[...]
    for sub in range(_TCBLK // 1024):
        b0 = sub * 1024
        y = jnp.concatenate(
            [x[:, b0 + c * 256:b0 + (c + 1) * 256] for c in range(4)], axis=0)
        o_ref[sub * 256:(sub + 1) * 256, :] = y.T


def _sc_body(idx1_hbm, idx2_hbm, emb4_hbm, reff_hbm, beta_hbm, out_hbm,
             idx1_v, idx2_v, k1_v, k2_v, o1_v, o2_v,
             zrow1_v, zrow2_v, r1_v, r2_v, beta_v, out_v, sem, rsem):
    wid = lax.axis_index("s") * _NC + lax.axis_index("c")
    base = wid * _BPW

    pltpu.sync_copy(idx1_hbm.at[pl.ds(base, _BPW)], idx1_v)
    pltpu.sync_copy(idx2_hbm.at[pl.ds(base, _BPW)], idx2_v)
    pltpu.sync_copy(beta_hbm, beta_v)

    rcopies = []
    for k in range(_NCHUNK):
        s = pl.ds(k * _CHUNK, _CHUNK)
        rcopies.append(pltpu.async_copy(reff_hbm.at[idx1_v.at[s]], r1_v.at[s], rsem))
        rcopies.append(pltpu.async_copy(reff_hbm.at[idx2_v.at[s]], r2_v.at[s], rsem))

    def prep(g, carry):
        s = pl.ds(g * _L, _L)
        i1 = idx1_v[s]
        i2 = idx2_v[s]
        k1_v[s] = ((i1 >> 10) << 8) + (i1 & 255)
        k2_v[s] = ((i2 >> 10) << 8) + (i2 & 255)
        o1_v[s] = ((i1 >> 8) & 3) << 5
        o2_v[s] = ((i2 >> 8) & 3) << 5
        return carry

    lax.fori_loop(0, _BPW // _L, prep, 0)

    beta = beta_v[...]
    lane = lax.iota(jnp.int32, _L)

    def issue(q):
        par = q & 1
        s = pl.ds(q * _CHUNK, _CHUNK)
        d = pl.ds(par * _CHUNK, _CHUNK)
        return [
            pltpu.async_copy(emb4_hbm.at[k1_v.at[s]], zrow1_v.at[d, :], sem),
            pltpu.async_copy(emb4_hbm.at[k2_v.at[s]], zrow2_v.at[d, :], sem),
        ]

    def compute_quarter(q):
        par = q & 1

        def group(g, carry2):
            sl = pl.ds(q * _CHUNK + g * _L, _L)
            rows = par * _CHUNK + g * _L + lane
            o1 = o1_v[sl]
            o2 = o2_v[sl]
            acc = jnp.zeros((_L,), jnp.float32)
            for d in range(_D):
                a = plsc.load_gather(zrow1_v, [rows, o1 + d])
                b = plsc.load_gather(zrow2_v, [rows, o2 + d])
                diff = a - b
                acc = acc + diff * diff
            i = plsc.bitcast(acc, jnp.int32)
            i = jnp.int32(0x5F3759DF) - (i >> 1)
            y = plsc.bitcast(i, jnp.float32)
            for _ in range(3):
                y = y * (1.5 - 0.5 * acc * y * y)
            dist = jnp.where(acc > 1e-35, acc * y, 0.0)
            out_v[sl] = r1_v[sl] + r2_v[sl] - beta * dist
            return carry2

        lax.fori_loop(0, _CHUNK // _L, group, 0)

    nq = _BPW // _CHUNK
    pending = issue(0)
    for c in rcopies:
        c.wait()
    for q in range(nq):
        for c in pending:
            c.wait()
        if q + 1 < nq:
            nxt = issue(q + 1)
        else:
            nxt = []
        compute_quarter(q)
        pending = nxt
    pltpu.sync_copy(out_v, out_hbm.at[pl.ds(base, _BPW)])


@jax.jit
def _run(p1, p2, embt, reff_flat, beta16):
    emb4 = pl.pallas_call(
        _tc_body,
        grid=(_TCGRID,),
        in_specs=[pl.BlockSpec((_D, _TCBLK), lambda j: (0, j))],
        out_specs=pl.BlockSpec((_TCBLK // 4, _RW), lambda j: (j, 0)),
        compiler_params=pltpu.CompilerParams(
            dimension_semantics=("arbitrary",)),
        out_shape=jax.ShapeDtypeStruct((_ROWS, _RW), jnp.float32),
    )(embt)

    ker = pl.kernel(
        _sc_body,
        out_type=jax.ShapeDtypeStruct((_B,), jnp.float32),
        mesh=plsc.VectorSubcoreMesh(core_axis_name="c", subcore_axis_name="s"),
        compiler_params=pltpu.CompilerParams(
            needs_layout_passes=False, use_tc_tiling_on_sc=True),
        scratch_types=[
            pltpu.VMEM((_BPW,), jnp.int32),
            pltpu.VMEM((_BPW,), jnp.int32),
            pltpu.VMEM((_BPW,), jnp.int32),
            pltpu.VMEM((_BPW,), jnp.int32),
            pltpu.VMEM((_BPW,), jnp.int32),
            pltpu.VMEM((_BPW,), jnp.int32),
            pltpu.VMEM((2 * _CHUNK, _RW), jnp.float32),
            pltpu.VMEM((2 * _CHUNK, _RW), jnp.float32),
            pltpu.VMEM((_BPW,), jnp.float32),
            pltpu.VMEM((_BPW,), jnp.float32),
            pltpu.VMEM((_L,), jnp.float32),
            pltpu.VMEM((_BPW,), jnp.float32),
            pltpu.SemaphoreType.DMA,
            pltpu.SemaphoreType.DMA,
        ],
    )
    return ker(p1, p2, emb4, reff_flat, beta16)


def kernel(protein1_idx, protein2_idx, embeddings, random_effects, beta):
    p1 = protein1_idx.astype(jnp.int32)
    p2 = protein2_idx.astype(jnp.int32)
    embt = embeddings.T
    reff_flat = random_effects.reshape(-1)
    beta16 = jnp.full((_L,), beta, jnp.float32)
    return _run(p1, p2, embt, reff_flat, beta16)


# speedup vs baseline: 3.9721x; 1.2013x over previous
import jax
import jax.numpy as jnp
from jax import lax
from jax.experimental import pallas as pl
from jax.experimental.pallas import tpu as pltpu
from jax.experimental.pallas import tpu_sc as plsc

_B = 16384
_D = 32
_L = 16
_RW = 128
_N = 1_000_000
_TCBLK = 16384
_TCGRID = -(-_N // _TCBLK)
_ROWS = _TCGRID * (_TCBLK // 4)

_INFO = plsc.get_sparse_core_info()
_NC = _INFO.num_cores
_NS = _INFO.num_subcores
_NW = _NC * _NS
_BPW = _B // _NW
_HALF = _BPW // 2
_HGROUPS = _HALF // _L
_CHUNK = 128
_NHCHUNK = _HALF // _CHUNK
_NCHUNK = _BPW // _CHUNK


def _tc_body(x_ref, o_ref):
    x = x_ref[...]
    for sub in range(_TCBLK // 1024):
        b0 = sub * 1024
        y = jnp.concatenate(
            [x[:, b0 + c * 256:b0 + (c + 1) * 256] for c in range(4)], axis=0)
        o_ref[sub * 256:(sub + 1) * 256, :] = y.T


def _sc_body(idx1_hbm, idx2_hbm, emb4_hbm, reff_hbm, beta_hbm, out_hbm,
             idx1_v, idx2_v, k1_v, k2_v, o1_v, o2_v,
             zrow1_v, zrow2_v, r1_v, r2_v, beta_v, out_v, sem, rsem):
    wid = lax.axis_index("s") * _NC + lax.axis_index("c")
    base = wid * _BPW

    pltpu.sync_copy(idx1_hbm.at[pl.ds(base, _BPW)], idx1_v)
    pltpu.sync_copy(idx2_hbm.at[pl.ds(base, _BPW)], idx2_v)
    pltpu.sync_copy(beta_hbm, beta_v)

    rcopies = []
    for k in range(_NCHUNK):
        s = pl.ds(k * _CHUNK, _CHUNK)
        rcopies.append(pltpu.async_copy(reff_hbm.at[idx1_v.at[s]], r1_v.at[s], rsem))
        rcopies.append(pltpu.async_copy(reff_hbm.at[idx2_v.at[s]], r2_v.at[s], rsem))

    def prep(g, carry):
        s = pl.ds(g * _L, _L)
        i1 = idx1_v[s]
        i2 = idx2_v[s]
        k1_v[s] = ((i1 >> 10) << 8) + (i1 & 255)
        k2_v[s] = ((i2 >> 10) << 8) + (i2 & 255)
        o1_v[s] = ((i1 >> 8) & 3) << 5
        o2_v[s] = ((i2 >> 8) & 3) << 5
        return carry

    lax.fori_loop(0, _BPW // _L, prep, 0)

    beta = beta_v[...]
    lane = lax.iota(jnp.int32, _L)

    def issue(q):
        par = q & 1
        s = pl.ds(q * _CHUNK, _CHUNK)
        d = pl.ds(par * _CHUNK, _CHUNK)
        return [
            pltpu.async_copy(emb4_hbm.at[k1_v.at[s]], zrow1_v.at[d, :], sem),
            pltpu.async_copy(emb4_hbm.at[k2_v.at[s]], zrow2_v.at[d, :], sem),
        ]

    def compute_quarter(q):
        par = q & 1

        def group(g, carry2):
            sl = pl.ds(q * _CHUNK + g * _L, _L)
            rows = par * _CHUNK + g * _L + lane
            o1 = o1_v[sl]
            o2 = o2_v[sl]
            acc = jnp.zeros((_L,), jnp.float32)
            for d in range(_D):
                a = plsc.load_gather(zrow1_v, [rows, o1 + d])
                b = plsc.load_gather(zrow2_v, [rows, o2 + d])
                diff = a - b
                acc = acc + diff * diff
            i = plsc.bitcast(acc, jnp.int32)
            i = jnp.int32(0x5F3759DF) - (i >> 1)
            y = plsc.bitcast(i, jnp.float32)
            for _ in range(3):
                y = y * (1.5 - 0.5 * acc * y * y)
            dist = jnp.where(acc > 1e-35, acc * y, 0.0)
            out_v[sl] = r1_v[sl] + r2_v[sl] - beta * dist
            return carry2

        lax.fori_loop(0, _CHUNK // _L, group, 0)

    nq = _BPW // _CHUNK
    pending = issue(0)
    for c in rcopies:
        c.wait()
    for q in range(nq):
        for c in pending:
            c.wait()
        if q + 1 < nq:
            nxt = issue(q + 1)
        else:
            nxt = []
        compute_quarter(q)
        pending = nxt
    pltpu.sync_copy(out_v, out_hbm.at[pl.ds(base, _BPW)])


@jax.jit
def _run(p1, p2, embt, reff_flat, beta16):
    emb4 = pl.pallas_call(
        _tc_body,
        grid=(_TCGRID,),
        in_specs=[pl.BlockSpec((_D, _TCBLK), lambda j: (0, j))],
        out_specs=pl.BlockSpec((_TCBLK // 4, _RW), lambda j: (j, 0)),
        compiler_params=pltpu.CompilerParams(
            dimension_semantics=("parallel",)),
        out_shape=jax.ShapeDtypeStruct((_ROWS, _RW), jnp.float32),
    )(embt)

    ker = pl.kernel(
        _sc_body,
        out_type=jax.ShapeDtypeStruct((_B,), jnp.float32),
        mesh=plsc.VectorSubcoreMesh(core_axis_name="c", subcore_axis_name="s"),
        compiler_params=pltpu.CompilerParams(
            needs_layout_passes=False, use_tc_tiling_on_sc=True),
        scratch_types=[
            pltpu.VMEM((_BPW,), jnp.int32),
            pltpu.VMEM((_BPW,), jnp.int32),
            pltpu.VMEM((_BPW,), jnp.int32),
            pltpu.VMEM((_BPW,), jnp.int32),
            pltpu.VMEM((_BPW,), jnp.int32),
            pltpu.VMEM((_BPW,), jnp.int32),
            pltpu.VMEM((2 * _CHUNK, _RW), jnp.float32),
            pltpu.VMEM((2 * _CHUNK, _RW), jnp.float32),
            pltpu.VMEM((_BPW,), jnp.float32),
            pltpu.VMEM((_BPW,), jnp.float32),
            pltpu.VMEM((_L,), jnp.float32),
            pltpu.VMEM((_BPW,), jnp.float32),
            pltpu.SemaphoreType.DMA,
            pltpu.SemaphoreType.DMA,
        ],
    )
    return ker(p1, p2, emb4, reff_flat, beta16)


def kernel(protein1_idx, protein2_idx, embeddings, random_effects, beta):
    p1 = protein1_idx.astype(jnp.int32)
    p2 = protein2_idx.astype(jnp.int32)
    embt = embeddings.T
    reff_flat = random_effects.reshape(-1)
    beta16 = jnp.full((_L,), beta, jnp.float32)
    return _run(p1, p2, embt, reff_flat, beta16)


# speedup vs baseline: 4.3470x; 1.0944x over previous
import jax
import jax.numpy as jnp
from jax import lax
from jax.experimental import pallas as pl
from jax.experimental.pallas import tpu as pltpu
from jax.experimental.pallas import tpu_sc as plsc

_B = 16384
_D = 32
_L = 16
_RW = 128
_N = 1_000_000
_TCBLK = 65536
_TCGRID = -(-_N // _TCBLK)
_ROWS = _TCGRID * (_TCBLK // 4)

_INFO = plsc.get_sparse_core_info()
_NC = _INFO.num_cores
_NS = _INFO.num_subcores
_NW = _NC * _NS
_BPW = _B // _NW
_HALF = _BPW // 2
_HGROUPS = _HALF // _L
_CHUNK = 128
_NHCHUNK = _HALF // _CHUNK
_NCHUNK = _BPW // _CHUNK


def _tc_body(x_ref, o_ref):
    x = x_ref[...]
    for sub in range(_TCBLK // 1024):
        b0 = sub * 1024
        y = jnp.concatenate(
            [x[:, b0 + c * 256:b0 + (c + 1) * 256] for c in range(4)], axis=0)
        o_ref[sub * 256:(sub + 1) * 256, :] = y.T


def _sc_body(idx1_hbm, idx2_hbm, emb4_hbm, reff_hbm, beta_hbm, out_hbm,
             idx1_v, idx2_v, k1_v, k2_v, o1_v, o2_v,
             zrow1_v, zrow2_v, r1_v, r2_v, beta_v, out_v, sem, rsem):
    wid = lax.axis_index("s") * _NC + lax.axis_index("c")
    base = wid * _BPW

    pltpu.sync_copy(idx1_hbm.at[pl.ds(base, _BPW)], idx1_v)
    pltpu.sync_copy(idx2_hbm.at[pl.ds(base, _BPW)], idx2_v)
    pltpu.sync_copy(beta_hbm, beta_v)

    rcopies = []
    for k in range(_NCHUNK):
        s = pl.ds(k * _CHUNK, _CHUNK)
        rcopies.append(pltpu.async_copy(reff_hbm.at[idx1_v.at[s]], r1_v.at[s], rsem))
        rcopies.append(pltpu.async_copy(reff_hbm.at[idx2_v.at[s]], r2_v.at[s], rsem))

    def prep(g, carry):
        s = pl.ds(g * _L, _L)
        i1 = idx1_v[s]
        i2 = idx2_v[s]
        k1_v[s] = ((i1 >> 10) << 8) + (i1 & 255)
        k2_v[s] = ((i2 >> 10) << 8) + (i2 & 255)
        o1_v[s] = ((i1 >> 8) & 3) << 5
        o2_v[s] = ((i2 >> 8) & 3) << 5
        return carry

    lax.fori_loop(0, _BPW // _L, prep, 0)

    beta = beta_v[...]
    lane = lax.iota(jnp.int32, _L)

    def issue(q):
        par = q & 1
        s = pl.ds(q * _CHUNK, _CHUNK)
        d = pl.ds(par * _CHUNK, _CHUNK)
        return [
            pltpu.async_copy(emb4_hbm.at[k1_v.at[s]], zrow1_v.at[d, :], sem),
            pltpu.async_copy(emb4_hbm.at[k2_v.at[s]], zrow2_v.at[d, :], sem),
        ]

    def compute_quarter(q):
        par = q & 1

        def group(g, carry2):
            sl = pl.ds(q * _CHUNK + g * _L, _L)
            rows = par * _CHUNK + g * _L + lane
            o1 = o1_v[sl]
            o2 = o2_v[sl]
            acc = jnp.zeros((_L,), jnp.float32)
            for d in range(_D):
                a = plsc.load_gather(zrow1_v, [rows, o1 + d])
                b = plsc.load_gather(zrow2_v, [rows, o2 + d])
                diff = a - b
                acc = acc + diff * diff
            i = plsc.bitcast(acc, jnp.int32)
            i = jnp.int32(0x5F3759DF) - (i >> 1)
            y = plsc.bitcast(i, jnp.float32)
            for _ in range(3):
                y = y * (1.5 - 0.5 * acc * y * y)
            dist = jnp.where(acc > 1e-35, acc * y, 0.0)
            out_v[sl] = r1_v[sl] + r2_v[sl] - beta * dist
            return carry2

        lax.fori_loop(0, _CHUNK // _L, group, 0)

    nq = _BPW // _CHUNK
    pending = issue(0)
    for c in rcopies:
        c.wait()
    for q in range(nq):
        for c in pending:
            c.wait()
        if q + 1 < nq:
            nxt = issue(q + 1)
        else:
            nxt = []
        compute_quarter(q)
        pending = nxt
    pltpu.sync_copy(out_v, out_hbm.at[pl.ds(base, _BPW)])


@jax.jit
def _run(p1, p2, embt, reff_flat, beta16):
    emb4 = pl.pallas_call(
        _tc_body,
        grid=(_TCGRID,),
        in_specs=[pl.BlockSpec((_D, _TCBLK), lambda j: (0, j))],
        out_specs=pl.BlockSpec((_TCBLK // 4, _RW), lambda j: (j, 0)),
        compiler_params=pltpu.CompilerParams(
            dimension_semantics=("parallel",)),
        out_shape=jax.ShapeDtypeStruct((_ROWS, _RW), jnp.float32),
    )(embt)

    ker = pl.kernel(
        _sc_body,
        out_type=jax.ShapeDtypeStruct((_B,), jnp.float32),
        mesh=plsc.VectorSubcoreMesh(core_axis_name="c", subcore_axis_name="s"),
        compiler_params=pltpu.CompilerParams(
            needs_layout_passes=False, use_tc_tiling_on_sc=True),
        scratch_types=[
            pltpu.VMEM((_BPW,), jnp.int32),
            pltpu.VMEM((_BPW,), jnp.int32),
            pltpu.VMEM((_BPW,), jnp.int32),
            pltpu.VMEM((_BPW,), jnp.int32),
            pltpu.VMEM((_BPW,), jnp.int32),
            pltpu.VMEM((_BPW,), jnp.int32),
            pltpu.VMEM((2 * _CHUNK, _RW), jnp.float32),
            pltpu.VMEM((2 * _CHUNK, _RW), jnp.float32),
            pltpu.VMEM((_BPW,), jnp.float32),
            pltpu.VMEM((_BPW,), jnp.float32),
            pltpu.VMEM((_L,), jnp.float32),
            pltpu.VMEM((_BPW,), jnp.float32),
            pltpu.SemaphoreType.DMA,
            pltpu.SemaphoreType.DMA,
        ],
    )
    return ker(p1, p2, emb4, reff_flat, beta16)


def kernel(protein1_idx, protein2_idx, embeddings, random_effects, beta):
    p1 = protein1_idx.astype(jnp.int32)
    p2 = protein2_idx.astype(jnp.int32)
    embt = embeddings.T
    reff_flat = random_effects.reshape(-1)
    beta16 = jnp.full((_L,), beta, jnp.float32)
    return _run(p1, p2, embt, reff_flat, beta16)
